# baseline (speedup 1.0000x reference)
import functools

import jax
import jax.numpy as jnp
from jax import lax
from jax.experimental import pallas as pl
from jax.experimental.pallas import tpu as pltpu
from jax.experimental.pallas import tpu_sc as plsc

NC = 2
NS = 16
NW = NC * NS
CHUNK = 80


def _silu(x):
    return x * jax.nn.sigmoid(x)


def _tc0_body(s_ref, wa_ref, wb_ref, b1_ref, p_ref, q_ref):
    s = s_ref[...]
    p_ref[...] = jnp.dot(s, wa_ref[...],
                         preferred_element_type=jnp.float32) + b1_ref[...]
    q_ref[...] = jnp.dot(s, wb_ref[...], preferred_element_type=jnp.float32)


def _sca_body(row_h, col_h, p_h, q_h, cx_h, cy_h, cz_h, wc_h,
              z_h, dx_h, dy_h, dz_h,
              rowi0, coli0, zb0, qb0, dxb0, dyb0, dzb0, d2b0,
              rowi1, coli1, zb1, qb1, dxb1, dyb1, dzb1, d2b1,
              cxv, cyv, czv, wcv,
              semi0, semg0, semw0, semi1, semg1, semw1, E=None):
    cid = lax.axis_index("c")
    sid = lax.axis_index("s")
    wid = sid * NC + cid
    epw = E // NW
    nch = epw // CHUNK
    base0 = pl.multiple_of(wid * epw, 8)
    S = [
        dict(rowi=rowi0, coli=coli0, zb=zb0, qb=qb0, dxb=dxb0, dyb=dyb0,
             dzb=dzb0, d2b=d2b0, semi=semi0, semg=semg0, semw=semw0),
        dict(rowi=rowi1, coli=coli1, zb=zb1, qb=qb1, dxb=dxb1, dyb=dyb1,
             dzb=dzb1, d2b=d2b1, semi=semi1, semg=semg1, semw=semw1),
    ]

    def bse(k):
        return pl.multiple_of(base0 + k * CHUNK, 8)

    def issue_idx(k, s):
        pltpu.async_copy(row_h.at[pl.ds(bse(k), CHUNK)], S[s]['rowi'],
                         S[s]['semi'])
        pltpu.async_copy(col_h.at[pl.ds(bse(k), CHUNK)], S[s]['coli'],
                         S[s]['semi'])

    def wait_idx(k, s):
        pltpu.make_async_copy(row_h.at[pl.ds(bse(k), CHUNK)], S[s]['rowi'],
                              S[s]['semi']).wait()
        pltpu.make_async_copy(col_h.at[pl.ds(bse(k), CHUNK)], S[s]['coli'],
                              S[s]['semi']).wait()

    def issue_gat(s):
        pltpu.async_copy(p_h.at[S[s]['rowi']], S[s]['zb'], S[s]['semg'])
        pltpu.async_copy(q_h.at[S[s]['coli']], S[s]['qb'], S[s]['semg'])

    def wait_gat(s):
        pltpu.make_async_copy(p_h.at[S[s]['rowi']], S[s]['zb'],
                              S[s]['semg']).wait()
        pltpu.make_async_copy(q_h.at[S[s]['coli']], S[s]['qb'],
                              S[s]['semg']).wait()

    def geometry(s):
        st = S[s]
        for g in range(CHUNK // 16):
            sl = pl.ds(g * 16, 16)
            rv = st['rowi'][sl]
            cv = st['coli'][sl]
            dx = plsc.load_gather(cxv, [rv]) - plsc.load_gather(cxv, [cv])
            dy = plsc.load_gather(cyv, [rv]) - plsc.load_gather(cyv, [cv])
            dz = plsc.load_gather(czv, [rv]) - plsc.load_gather(czv, [cv])
            st['dxb'][sl] = dx
            st['dyb'][sl] = dy
            st['dzb'][sl] = dz
            st['d2b'][sl] = dx * dx + dy * dy + dz * dz

    def add_q(s):
        st = S[s]

        def add_row(e, c):
            idx = jnp.full((16,), e, jnp.int32)
            d2bc = plsc.load_gather(st['d2b'], [idx])
            for j in range(8):
                sl = pl.ds(j * 16, 16)
                st['zb'][e, sl] = (st['zb'][e, sl] + st['qb'][e, sl]
                                   + d2bc * wcv[sl])
            return c
        lax.fori_loop(0, CHUNK, add_row, 0, unroll=2)

    def issue_outs(k, s):
        st = S[s]
        pltpu.async_copy(st['zb'], z_h.at[pl.ds(bse(k), CHUNK), :], st['semw'])
        pltpu.async_copy(st['dxb'], dx_h.at[pl.ds(bse(k), CHUNK)], st['semw'])
        pltpu.async_copy(st['dyb'], dy_h.at[pl.ds(bse(k), CHUNK)], st['semw'])
        pltpu.async_copy(st['dzb'], dz_h.at[pl.ds(bse(k), CHUNK)], st['semw'])

    def wait_outs(k, s):
        st = S[s]
        pltpu.make_async_copy(st['zb'], z_h.at[pl.ds(bse(k), CHUNK), :],
                              st['semw']).wait()
        pltpu.make_async_copy(st['dxb'], dx_h.at[pl.ds(bse(k), CHUNK)],
                              st['semw']).wait()
        pltpu.make_async_copy(st['dyb'], dy_h.at[pl.ds(bse(k), CHUNK)],
                              st['semw']).wait()
        pltpu.make_async_copy(st['dzb'], dz_h.at[pl.ds(bse(k), CHUNK)],
                              st['semw']).wait()

    pltpu.sync_copy(cx_h, cxv)
    pltpu.sync_copy(cy_h, cyv)
    pltpu.sync_copy(cz_h, czv)
    pltpu.sync_copy(wc_h, wcv)

    issue_idx(0, 0)
    wait_idx(0, 0)
    issue_gat(0)
    issue_idx(1, 1)

    def pair_body(j, carry):
        for b in range(2):
            k = 2 * j + b
            s = b
            o = 1 - b
            wait_idx(k + 1, o)

            @pl.when(k >= 1)
            def _():
                wait_outs(k - 1, o)
            issue_gat(o)
            geometry(s)
            wait_gat(s)

            @pl.when(k <= nch - 3)
            def _():
                issue_idx(k + 2, s)
            add_q(s)
            issue_outs(k, s)
        return carry

    lax.fori_loop(0, (nch - 1) // 2, pair_body, 0)

    k = nch - 1
    wait_outs(k - 1, 1)
    geometry(0)
    wait_gat(0)
    add_q(0)
    issue_outs(k, 0)
    wait_outs(k, 0)


def _tcb_body(z_ref, wm2_ref, b2_ref,
              wt1_ref, bt1_ref, wt2_ref, bt2_ref, wt3_ref,
              msg_ref, t_ref):
    bf16 = jnp.bfloat16
    u = _silu(z_ref[...])
    m = _silu(jnp.dot(u.astype(bf16), wm2_ref[...],
                      preferred_element_type=jnp.float32) + b2_ref[...])
    t1 = _silu(jnp.dot(m.astype(bf16), wt1_ref[...],
                       preferred_element_type=jnp.float32) + bt1_ref[...])
    t2 = _silu(jnp.dot(t1.astype(bf16), wt2_ref[...],
                       preferred_element_type=jnp.float32) + bt2_ref[...])
    msg_ref[...] = m
    tv = jnp.sum(t2 * wt3_ref[...], axis=1)
    tr = t2.shape[0] // 128
    i = pl.program_id(0)
    t_ref[pl.ds(i * tr, tr), :] = jnp.reshape(tv, (tr, 128))


def _scc_body(row_h, msg_h, t_h, dx_h, dy_h, dz_h,
              msp_h, np4_h,
              rowi0, mb0, tb0, dxb0, dyb0, dzb0, txb0, tyb0, tzb0,
              semr0, semw0,
              rowi1, mb1, tb1, dxb1, dyb1, dzb1, txb1, tyb1, tzb1,
              semr1, semw1,
              rowi2, mb2, tb2, dxb2, dyb2, dzb2, txb2, tyb2, tzb2,
              semr2, semw2,
              onesb, zrow, znum, stg, msum, nx_s, ny_s, nz_s, cnt_s,
              E=None, N=None, NPAD=None):
    S = [
        dict(rowi=rowi0, mb=mb0, tb=tb0, dxb=dxb0, dyb=dyb0, dzb=dzb0,
             txb=txb0, tyb=tyb0, tzb=tzb0, semr=semr0, semw=semw0),
        dict(rowi=rowi1, mb=mb1, tb=tb1, dxb=dxb1, dyb=dyb1, dzb=dzb1,
             txb=txb1, tyb=tyb1, tzb=tzb1, semr=semr1, semw=semw1),
        dict(rowi=rowi2, mb=mb2, tb=tb2, dxb=dxb2, dyb=dyb2, dzb=dzb2,
             txb=txb2, tyb=tyb2, tzb=tzb2, semr=semr2, semw=semw2),
    ]
    cid = lax.axis_index("c")
    sid = lax.axis_index("s")
    wid = sid * NC + cid
    epw = E // NW
    nch = epw // CHUNK
    base0 = pl.multiple_of(wid * epw, 8)
    c1 = (N // NS // 8) * 8
    tail = N - c1 * NS
    off1 = pl.multiple_of(sid * c1, 8)

    zv = jnp.zeros((16,), jnp.float32)

    def zr_body(r, c):
        for j in range(8):
            zrow[r, pl.ds(j * 16, 16)] = zv
        return c
    lax.fori_loop(0, zrow.shape[0], zr_body, 0, unroll=2)

    def zn_body(r, c):
        znum[pl.ds(r * 16, 16)] = zv
        return c
    lax.fori_loop(0, znum.shape[0] // 16, zn_body, 0, unroll=2)

    zr = zrow.shape[0]
    for k in range(c1 // zr):
        pltpu.sync_copy(zrow, msum.at[pl.ds(off1 + k * zr, zr), :])
    for acc in (nx_s, ny_s, nz_s, cnt_s):
        pltpu.sync_copy(znum.at[pl.ds(0, c1)], acc.at[pl.ds(off1, c1)])

    if tail > 0:
        @pl.when(sid == 0)
        def _():
            t0 = pl.multiple_of(c1 * NS, 8)
            pltpu.sync_copy(zrow.at[pl.ds(0, tail), :],
                            msum.at[pl.ds(t0, tail), :])
            for acc in (nx_s, ny_s, nz_s, cnt_s):
                pltpu.sync_copy(znum.at[pl.ds(0, tail)],
                                acc.at[pl.ds(t0, tail)])

    def ob_body(r, c):
        onesb[pl.ds(r * 16, 16)] = jnp.ones((16,), jnp.float32)
        return c
    lax.fori_loop(0, CHUNK // 16, ob_body, 0)

    plsc.subcore_barrier()

    def bse(k):
        return pl.multiple_of(base0 + k * CHUNK, 8)

    def issue_reads(k, s):
        st = S[s]
        pltpu.async_copy(row_h.at[pl.ds(bse(k), CHUNK)], st['rowi'],
                         st['semr'])
        pltpu.async_copy(msg_h.at[pl.ds(bse(k), CHUNK), :], st['mb'],
                         st['semr'])
        pltpu.async_copy(t_h.at[pl.ds(bse(k), CHUNK)], st['tb'], st['semr'])
        pltpu.async_copy(dx_h.at[pl.ds(bse(k), CHUNK)], st['dxb'], st['semr'])
        pltpu.async_copy(dy_h.at[pl.ds(bse(k), CHUNK)], st['dyb'], st['semr'])
        pltpu.async_copy(dz_h.at[pl.ds(bse(k), CHUNK)], st['dzb'], st['semr'])

    def wait_reads(k, s):
        st = S[s]
        pltpu.make_async_copy(row_h.at[pl.ds(bse(k), CHUNK)], st['rowi'],
                              st['semr']).wait()
        pltpu.make_async_copy(msg_h.at[pl.ds(bse(k), CHUNK), :], st['mb'],
                              st['semr']).wait()
        pltpu.make_async_copy(t_h.at[pl.ds(bse(k), CHUNK)], st['tb'],
                              st['semr']).wait()
        pltpu.make_async_copy(dx_h.at[pl.ds(bse(k), CHUNK)], st['dxb'],
                              st['semr']).wait()
        pltpu.make_async_copy(dy_h.at[pl.ds(bse(k), CHUNK)], st['dyb'],
                              st['semr']).wait()
        pltpu.make_async_copy(dz_h.at[pl.ds(bse(k), CHUNK)], st['dzb'],
                              st['semr']).wait()

    def compute(s):
        st = S[s]
        for g in range(CHUNK // 16):
            sl = pl.ds(g * 16, 16)
            tv = st['tb'][sl]
            st['txb'][sl] = tv * st['dxb'][sl]
            st['tyb'][sl] = tv * st['dyb'][sl]
            st['tzb'][sl] = tv * st['dzb'][sl]

    def issue_scat(s):
        st = S[s]
        pltpu.async_copy(st['mb'], msum.at[st['rowi']], st['semw'], add=True)
        pltpu.async_copy(st['txb'], nx_s.at[st['rowi']], st['semw'], add=True)
        pltpu.async_copy(st['tyb'], ny_s.at[st['rowi']], st['semw'], add=True)
        pltpu.async_copy(st['tzb'], nz_s.at[st['rowi']], st['semw'], add=True)
        pltpu.async_copy(onesb, cnt_s.at[st['rowi']], st['semw'], add=True)

    def wait_scat(s):
        st = S[s]
        pltpu.make_async_copy(st['mb'], msum.at[st['rowi']],
                              st['semw']).wait()
        pltpu.make_async_copy(st['txb'], nx_s.at[st['rowi']],
                              st['semw']).wait()
        pltpu.make_async_copy(st['tyb'], ny_s.at[st['rowi']],
                              st['semw']).wait()
        pltpu.make_async_copy(st['tzb'], nz_s.at[st['rowi']],
                              st['semw']).wait()
        pltpu.make_async_copy(onesb, cnt_s.at[st['rowi']],
                              st['semw']).wait()

    issue_reads(0, 0)

    def trip_body(j, carry):
        for b in range(3):
            k = 3 * j + b
            s = b

            @pl.when(k < nch)
            def _():
                @pl.when(k >= 2)
                def _():
                    wait_scat((b + 1) % 3)

                @pl.when(k + 1 < nch)
                def _():
                    issue_reads(k + 1, (b + 1) % 3)
                wait_reads(k, s)
                compute(s)
                issue_scat(s)
        return carry

    lax.fori_loop(0, (nch + 2) // 3, trip_body, 0)
    wait_scat((nch - 2) % 3)
    wait_scat((nch - 1) % 3)

    plsc.subcore_barrier()

    pltpu.sync_copy(msum.at[pl.ds(off1, c1), :],
                    msp_h.at[cid, pl.ds(off1, c1), :])
    for k, acc in enumerate((nx_s, ny_s, nz_s, cnt_s)):
        rb = pl.multiple_of((cid * 4 + k) * NPAD + off1, 8)
        pltpu.sync_copy(acc.at[pl.ds(off1, c1)], stg.at[pl.ds(0, c1)])
        pltpu.sync_copy(stg.at[pl.ds(0, c1)], np4_h.at[pl.ds(rb, c1)])

    @pl.when(sid == 0)
    def _():
        t0 = pl.multiple_of(c1 * NS, 8)
        if tail > 0:
            pltpu.sync_copy(msum.at[pl.ds(t0, tail), :],
                            msp_h.at[cid, pl.ds(t0, tail), :])
        for k, acc in enumerate((nx_s, ny_s, nz_s, cnt_s)):
            rb0 = pl.multiple_of((cid * 4 + k) * NPAD, 8)
            if tail > 0:
                pltpu.sync_copy(acc.at[pl.ds(t0, tail)], stg.at[pl.ds(0, tail)])
                pltpu.sync_copy(stg.at[pl.ds(0, tail)],
                                np4_h.at[pl.ds(rb0 + t0, tail)])
            if NPAD > N:
                pltpu.sync_copy(znum.at[pl.ds(0, NPAD - N)],
                                np4_h.at[pl.ds(rb0 + N, NPAD - N)])


def _tcd_body(s_ref, msp_ref, ct_ref, np4_ref,
              wp1a_ref, wp1b_ref, bp1_ref, wp2_ref, bp2_ref,
              so_ref, co_ref):
    s = s_ref[...]
    ms = msp_ref[0] + msp_ref[1]
    p = _silu(jnp.dot(s, wp1a_ref[...], preferred_element_type=jnp.float32)
              + jnp.dot(ms, wp1b_ref[...], preferred_element_type=jnp.float32)
              + bp1_ref[...])
    so_ref[...] = s + jnp.dot(p, wp2_ref[...],
                              preferred_element_type=jnp.float32) + bp2_ref[...]
    nps = np4_ref[0] + np4_ref[1]
    cnt = nps[3:4, :]
    co_ref[...] = ct_ref[...] + nps / jnp.clip(cnt, 1.0, None)


def kernel(edge_index, str_feature, coord_feature, W_m1, b_m1, W_m2, b_m2,
           W_t1, b_t1, W_t2, b_t2, W_t3, W_p1, b_p1, W_p2, b_p2):
    N, raw = str_feature.shape
    E = edge_index.shape[1]
    hid = W_m2.shape[0]
    f32 = jnp.float32
    NPAD = 10240
    assert E % (NW * CHUNK) == 0 and N % NS == 0

    row = edge_index[0]
    col = edge_index[1]
    cx = coord_feature[:, 0]
    cy = coord_feature[:, 1]
    cz = coord_feature[:, 2]

    W_aT = W_m1[:, :raw].T
    W_bT = W_m1[:, raw:2 * raw].T
    wc_row = W_m1[:, 2 * raw].reshape(1, hid)
    b1_row = b_m1.reshape(1, hid)
    Wm2T = W_m2.T.astype(jnp.bfloat16)
    b2_row = b_m2.reshape(1, hid)
    Wt1T = W_t1.T.astype(jnp.bfloat16)
    bt1_row = b_t1.reshape(1, hid)
    Wt2T = W_t2.T.astype(jnp.bfloat16)
    bt2_row = b_t2.reshape(1, hid)
    wt3_row = W_t3.reshape(1, hid)
    Wp1aT = W_p1[:, :raw].T
    Wp1bT = W_p1[:, raw:].T
    bp1_row = b_p1.reshape(1, hid)
    Wp2T = W_p2.T
    bp2_row = b_p2.reshape(1, raw)

    P, Q = pl.pallas_call(
        _tc0_body,
        out_shape=(jax.ShapeDtypeStruct((N, hid), f32),
                   jax.ShapeDtypeStruct((N, hid), f32)),
    )(str_feature, W_aT, W_bT, b1_row)

    mesh = plsc.VectorSubcoreMesh(core_axis_name="c", subcore_axis_name="s")
    sca = pl.kernel(
        functools.partial(_sca_body, E=E),
        compiler_params=pltpu.CompilerParams(needs_layout_passes=False),
        out_type=(jax.ShapeDtypeStruct((E, hid), f32),
                  jax.ShapeDtypeStruct((E,), f32),
                  jax.ShapeDtypeStruct((E,), f32),
                  jax.ShapeDtypeStruct((E,), f32)),
        mesh=mesh,
        scratch_types=(
            [pltpu.VMEM((CHUNK,), jnp.int32),
             pltpu.VMEM((CHUNK,), jnp.int32),
             pltpu.VMEM((CHUNK, hid), f32),
             pltpu.VMEM((CHUNK, hid), f32),
             pltpu.VMEM((CHUNK,), f32),
             pltpu.VMEM((CHUNK,), f32),
             pltpu.VMEM((CHUNK,), f32),
             pltpu.VMEM((CHUNK,), f32)] * 2
            + [pltpu.VMEM((N,), f32),
               pltpu.VMEM((N,), f32),
               pltpu.VMEM((N,), f32),
               pltpu.VMEM((hid,), f32)]
            + [pltpu.SemaphoreType.DMA] * 6
        ),
    )
    z, dxe, dye, dze = sca(row, col, P, Q, cx, cy, cz,
                           W_m1[:, 2 * raw].reshape(hid))

    BE = 2560
    nb = E // BE
    tr = BE // 128
    msg, t2d = pl.pallas_call(
        _tcb_body,
        grid=(nb,),
        in_specs=[
            pl.BlockSpec((BE, hid), lambda i: (i, 0)),
            pl.BlockSpec((hid, hid), lambda i: (0, 0)),
            pl.BlockSpec((1, hid), lambda i: (0, 0)),
            pl.BlockSpec((hid, hid), lambda i: (0, 0)),
            pl.BlockSpec((1, hid), lambda i: (0, 0)),
            pl.BlockSpec((hid, hid), lambda i: (0, 0)),
            pl.BlockSpec((1, hid), lambda i: (0, 0)),
            pl.BlockSpec((1, hid), lambda i: (0, 0)),
        ],
        out_specs=[
            pl.BlockSpec((BE, hid), lambda i: (i, 0)),
            pl.BlockSpec((E // 128, 128), lambda i: (0, 0)),
        ],
        out_shape=(jax.ShapeDtypeStruct((E, hid), f32),
                   jax.ShapeDtypeStruct((E // 128, 128), f32)),
    )(z, Wm2T, b2_row,
      Wt1T, bt1_row, Wt2T, bt2_row, wt3_row)

    scc = pl.kernel(
        functools.partial(_scc_body, E=E, N=N, NPAD=NPAD),
        out_type=(jax.ShapeDtypeStruct((NC, N, hid), f32),
                  jax.ShapeDtypeStruct((NC * 4 * NPAD,), f32)),
        mesh=mesh,
        scratch_types=(
            [pltpu.VMEM((CHUNK,), jnp.int32),
             pltpu.VMEM((CHUNK, hid), f32),
             pltpu.VMEM((CHUNK,), f32),
             pltpu.VMEM((CHUNK,), f32),
             pltpu.VMEM((CHUNK,), f32),
             pltpu.VMEM((CHUNK,), f32),
             pltpu.VMEM((CHUNK,), f32),
             pltpu.VMEM((CHUNK,), f32),
             pltpu.VMEM((CHUNK,), f32),
             pltpu.SemaphoreType.DMA,
             pltpu.SemaphoreType.DMA] * 3
            + [pltpu.VMEM((CHUNK,), f32),
               pltpu.VMEM((104, hid), f32),
               pltpu.VMEM((640,), f32),
               pltpu.VMEM((640,), f32),
               pltpu.VMEM_SHARED((N, hid), f32),
               pltpu.VMEM_SHARED((N,), f32),
               pltpu.VMEM_SHARED((N,), f32),
               pltpu.VMEM_SHARED((N,), f32),
               pltpu.VMEM_SHARED((N,), f32)]
        ),
    )
    msp, np4 = scc(row, msg, t2d.reshape(E), dxe, dye, dze)
    np4 = np4.reshape(NC, 4, NPAD)

    coordT4 = jnp.zeros((4, NPAD), f32).at[:3, :N].set(coord_feature.T)
    BN = 2000
    BNP = NPAD // (N // BN)
    str_out, coT = pl.pallas_call(
        _tcd_body,
        grid=(N // BN,),
        in_specs=[
            pl.BlockSpec((BN, hid), lambda i: (i, 0)),
            pl.BlockSpec((NC, BN, hid), lambda i: (0, i, 0)),
            pl.BlockSpec((4, BNP), lambda i: (0, i)),
            pl.BlockSpec((NC, 4, BNP), lambda i: (0, 0, i)),
            pl.BlockSpec((raw, hid), lambda i: (0, 0)),
            pl.BlockSpec((hid, hid), lambda i: (0, 0)),
            pl.BlockSpec((1, hid), lambda i: (0, 0)),
            pl.BlockSpec((hid, raw), lambda i: (0, 0)),
            pl.BlockSpec((1, raw), lambda i: (0, 0)),
        ],
        out_specs=[
            pl.BlockSpec((BN, raw), lambda i: (i, 0)),
            pl.BlockSpec((4, BNP), lambda i: (0, i)),
        ],
        out_shape=(jax.ShapeDtypeStruct((N, raw), f32),
                   jax.ShapeDtypeStruct((4, NPAD), f32)),
    )(str_feature, msp, coordT4, np4, Wp1aT, Wp1bT, bp1_row, Wp2T, bp2_row)

    coord_out = coT[:3, :N].T
    return str_out, coord_out


# speedup vs baseline: 1.5805x; 1.5805x over previous
import functools

import jax
import jax.numpy as jnp
from jax import lax
from jax.experimental import pallas as pl
from jax.experimental.pallas import tpu as pltpu
from jax.experimental.pallas import tpu_sc as plsc

NC = 2
NS = 16
NW = NC * NS
CHUNK = 80


def _silu(x):
    return x * jax.nn.sigmoid(x)


def _tc0_body(s_ref, wa_ref, wb_ref, b1_ref, p_ref, q_ref):
    s = s_ref[...]
    p_ref[...] = jnp.dot(s, wa_ref[...],
                         preferred_element_type=jnp.float32) + b1_ref[...]
    q_ref[...] = jnp.dot(s, wb_ref[...], preferred_element_type=jnp.float32)


def _sca_body(row_h, col_h, p_h, q_h, cx_h, cy_h, cz_h,
              z_h, dx_h, dy_h, dz_h, d2_h,
              rowi0, coli0, zb0, qb0, dxb0, dyb0, dzb0, d2b0,
              rowi1, coli1, zb1, qb1, dxb1, dyb1, dzb1, d2b1,
              cxv, cyv, czv,
              semi0, semg0, semw0, semi1, semg1, semw1, E=None):
    cid = lax.axis_index("c")
    sid = lax.axis_index("s")
    wid = sid * NC + cid
    epw = E // NW
    nch = epw // CHUNK
    base0 = pl.multiple_of(wid * epw, 8)
    S = [
        dict(rowi=rowi0, coli=coli0, zb=zb0, qb=qb0, dxb=dxb0, dyb=dyb0,
             dzb=dzb0, d2b=d2b0, semi=semi0, semg=semg0, semw=semw0),
        dict(rowi=rowi1, coli=coli1, zb=zb1, qb=qb1, dxb=dxb1, dyb=dyb1,
             dzb=dzb1, d2b=d2b1, semi=semi1, semg=semg1, semw=semw1),
    ]

    def bse(k):
        return pl.multiple_of(base0 + k * CHUNK, 8)

    def issue_idx(k, s):
        pltpu.async_copy(row_h.at[pl.ds(bse(k), CHUNK)], S[s]['rowi'],
                         S[s]['semi'])
        pltpu.async_copy(col_h.at[pl.ds(bse(k), CHUNK)], S[s]['coli'],
                         S[s]['semi'])

    def wait_idx(k, s):
        pltpu.make_async_copy(row_h.at[pl.ds(bse(k), CHUNK)], S[s]['rowi'],
                              S[s]['semi']).wait()
        pltpu.make_async_copy(col_h.at[pl.ds(bse(k), CHUNK)], S[s]['coli'],
                              S[s]['semi']).wait()

    def issue_gat(s):
        pltpu.async_copy(p_h.at[S[s]['rowi']], S[s]['zb'], S[s]['semg'])
        pltpu.async_copy(q_h.at[S[s]['coli']], S[s]['qb'], S[s]['semg'])

    def wait_gat(s):
        pltpu.make_async_copy(p_h.at[S[s]['rowi']], S[s]['zb'],
                              S[s]['semg']).wait()
        pltpu.make_async_copy(q_h.at[S[s]['coli']], S[s]['qb'],
                              S[s]['semg']).wait()

    def geometry(s):
        st = S[s]
        for g in range(CHUNK // 16):
            sl = pl.ds(g * 16, 16)
            rv = st['rowi'][sl]
            cv = st['coli'][sl]
            dx = plsc.load_gather(cxv, [rv]) - plsc.load_gather(cxv, [cv])
            dy = plsc.load_gather(cyv, [rv]) - plsc.load_gather(cyv, [cv])
            dz = plsc.load_gather(czv, [rv]) - plsc.load_gather(czv, [cv])
            st['dxb'][sl] = dx
            st['dyb'][sl] = dy
            st['dzb'][sl] = dz
            st['d2b'][sl] = dx * dx + dy * dy + dz * dz

    def add_q(s):
        st = S[s]

        def add_row(e, c):
            for j in range(8):
                plsc.addupdate(st['zb'].at[e, pl.ds(j * 16, 16)],
                               st['qb'][e, pl.ds(j * 16, 16)])
            return c
        lax.fori_loop(0, CHUNK, add_row, 0, unroll=2)

    def issue_outs(k, s):
        st = S[s]
        pltpu.async_copy(st['zb'], z_h.at[pl.ds(bse(k), CHUNK), :], st['semw'])
        pltpu.async_copy(st['dxb'], dx_h.at[pl.ds(bse(k), CHUNK)], st['semw'])
        pltpu.async_copy(st['dyb'], dy_h.at[pl.ds(bse(k), CHUNK)], st['semw'])
        pltpu.async_copy(st['dzb'], dz_h.at[pl.ds(bse(k), CHUNK)], st['semw'])
        pltpu.async_copy(st['d2b'], d2_h.at[pl.ds(bse(k), CHUNK)], st['semw'])

    def wait_outs(k, s):
        st = S[s]
        pltpu.make_async_copy(st['zb'], z_h.at[pl.ds(bse(k), CHUNK), :],
                              st['semw']).wait()
        pltpu.make_async_copy(st['dxb'], dx_h.at[pl.ds(bse(k), CHUNK)],
                              st['semw']).wait()
        pltpu.make_async_copy(st['dyb'], dy_h.at[pl.ds(bse(k), CHUNK)],
                              st['semw']).wait()
        pltpu.make_async_copy(st['dzb'], dz_h.at[pl.ds(bse(k), CHUNK)],
                              st['semw']).wait()
        pltpu.make_async_copy(st['d2b'], d2_h.at[pl.ds(bse(k), CHUNK)],
                              st['semw']).wait()

    pltpu.sync_copy(cx_h, cxv)
    pltpu.sync_copy(cy_h, cyv)
    pltpu.sync_copy(cz_h, czv)

    issue_idx(0, 0)
    wait_idx(0, 0)
    issue_gat(0)
    issue_idx(1, 1)

    def pair_body(j, carry):
        for b in range(2):
            k = 2 * j + b
            s = b
            o = 1 - b
            wait_idx(k + 1, o)

            @pl.when(k >= 1)
            def _():
                wait_outs(k - 1, o)
            issue_gat(o)
            geometry(s)
            wait_gat(s)

            @pl.when(k <= nch - 3)
            def _():
                issue_idx(k + 2, s)
            add_q(s)
            issue_outs(k, s)
        return carry

    lax.fori_loop(0, (nch - 1) // 2, pair_body, 0)

    k = nch - 1
    wait_outs(k - 1, 1)
    geometry(0)
    wait_gat(0)
    add_q(0)
    issue_outs(k, 0)
    wait_outs(k, 0)


def _tcb_body(z_ref, d2_ref, wc_ref, wm2_ref, b2_ref,
              wt1_ref, bt1_ref, wt2_ref, bt2_ref, wt3_ref,
              msg_ref, t_ref):
    bf16 = jnp.bfloat16
    ntr = z_ref.shape[0] // 128
    ib = pl.program_id(0)
    wc = wc_ref[...]
    rows = []
    for r in range(ntr):
        d2r = d2_ref[pl.ds(ib * ntr + r, 1), :]
        rows.append(lax.dot_general(d2r, wc, (((0,), (0,)), ((), ())),
                                    preferred_element_type=jnp.float32))
    u = _silu(z_ref[...] + jnp.concatenate(rows, axis=0))
    m = _silu(jnp.dot(u.astype(bf16), wm2_ref[...],
                      preferred_element_type=jnp.float32) + b2_ref[...])
    t1 = _silu(jnp.dot(m.astype(bf16), wt1_ref[...],
                       preferred_element_type=jnp.float32) + bt1_ref[...])
    t2 = _silu(jnp.dot(t1.astype(bf16), wt2_ref[...],
                       preferred_element_type=jnp.float32) + bt2_ref[...])
    msg_ref[...] = m
    tv = jnp.sum(t2 * wt3_ref[...], axis=1)
    tr = t2.shape[0] // 128
    i = pl.program_id(0)
    t_ref[pl.ds(i * tr, tr), :] = jnp.reshape(tv, (tr, 128))


def _scc_body(row_h, msg_h, t_h, dx_h, dy_h, dz_h,
              msp_h, np4_h,
              rowi0, mb0, tb0, dxb0, dyb0, dzb0, txb0, tyb0, tzb0,
              semr0, semw0,
              rowi1, mb1, tb1, dxb1, dyb1, dzb1, txb1, tyb1, tzb1,
              semr1, semw1,
              rowi2, mb2, tb2, dxb2, dyb2, dzb2, txb2, tyb2, tzb2,
              semr2, semw2,
              onesb, zrow, znum, stg, msum, nx_s, ny_s, nz_s, cnt_s,
              E=None, N=None, NPAD=None):
    S = [
        dict(rowi=rowi0, mb=mb0, tb=tb0, dxb=dxb0, dyb=dyb0, dzb=dzb0,
             txb=txb0, tyb=tyb0, tzb=tzb0, semr=semr0, semw=semw0),
        dict(rowi=rowi1, mb=mb1, tb=tb1, dxb=dxb1, dyb=dyb1, dzb=dzb1,
             txb=txb1, tyb=tyb1, tzb=tzb1, semr=semr1, semw=semw1),
        dict(rowi=rowi2, mb=mb2, tb=tb2, dxb=dxb2, dyb=dyb2, dzb=dzb2,
             txb=txb2, tyb=tyb2, tzb=tzb2, semr=semr2, semw=semw2),
    ]
    cid = lax.axis_index("c")
    sid = lax.axis_index("s")
    wid = sid * NC + cid
    epw = E // NW
    nch = epw // CHUNK
    base0 = pl.multiple_of(wid * epw, 8)
    c1 = (N // NS // 8) * 8
    tail = N - c1 * NS
    off1 = pl.multiple_of(sid * c1, 8)

    zv = jnp.zeros((16,), jnp.float32)

    def zr_body(r, c):
        for j in range(8):
            zrow[r, pl.ds(j * 16, 16)] = zv
        return c
    lax.fori_loop(0, zrow.shape[0], zr_body, 0, unroll=2)

    def zn_body(r, c):
        znum[pl.ds(r * 16, 16)] = zv
        return c
    lax.fori_loop(0, znum.shape[0] // 16, zn_body, 0, unroll=2)

    zr = zrow.shape[0]
    for k in range(c1 // zr):
        pltpu.sync_copy(zrow, msum.at[pl.ds(off1 + k * zr, zr), :])
    for acc in (nx_s, ny_s, nz_s, cnt_s):
        pltpu.sync_copy(znum.at[pl.ds(0, c1)], acc.at[pl.ds(off1, c1)])

    if tail > 0:
        @pl.when(sid == 0)
        def _():
            t0 = pl.multiple_of(c1 * NS, 8)
            pltpu.sync_copy(zrow.at[pl.ds(0, tail), :],
                            msum.at[pl.ds(t0, tail), :])
            for acc in (nx_s, ny_s, nz_s, cnt_s):
                pltpu.sync_copy(znum.at[pl.ds(0, tail)],
                                acc.at[pl.ds(t0, tail)])

    def ob_body(r, c):
        onesb[pl.ds(r * 16, 16)] = jnp.ones((16,), jnp.float32)
        return c
    lax.fori_loop(0, CHUNK // 16, ob_body, 0)

    plsc.subcore_barrier()

    def bse(k):
        return pl.multiple_of(base0 + k * CHUNK, 8)

    def issue_reads(k, s):
        st = S[s]
        pltpu.async_copy(row_h.at[pl.ds(bse(k), CHUNK)], st['rowi'],
                         st['semr'])
        pltpu.async_copy(msg_h.at[pl.ds(bse(k), CHUNK), :], st['mb'],
                         st['semr'])
        pltpu.async_copy(t_h.at[pl.ds(bse(k), CHUNK)], st['tb'], st['semr'])
        pltpu.async_copy(dx_h.at[pl.ds(bse(k), CHUNK)], st['dxb'], st['semr'])
        pltpu.async_copy(dy_h.at[pl.ds(bse(k), CHUNK)], st['dyb'], st['semr'])
        pltpu.async_copy(dz_h.at[pl.ds(bse(k), CHUNK)], st['dzb'], st['semr'])

    def wait_reads(k, s):
        st = S[s]
        pltpu.make_async_copy(row_h.at[pl.ds(bse(k), CHUNK)], st['rowi'],
                              st['semr']).wait()
        pltpu.make_async_copy(msg_h.at[pl.ds(bse(k), CHUNK), :], st['mb'],
                              st['semr']).wait()
        pltpu.make_async_copy(t_h.at[pl.ds(bse(k), CHUNK)], st['tb'],
                              st['semr']).wait()
        pltpu.make_async_copy(dx_h.at[pl.ds(bse(k), CHUNK)], st['dxb'],
                              st['semr']).wait()
        pltpu.make_async_copy(dy_h.at[pl.ds(bse(k), CHUNK)], st['dyb'],
                              st['semr']).wait()
        pltpu.make_async_copy(dz_h.at[pl.ds(bse(k), CHUNK)], st['dzb'],
                              st['semr']).wait()

    def compute(s):
        st = S[s]
        for g in range(CHUNK // 16):
            sl = pl.ds(g * 16, 16)
            tv = st['tb'][sl]
            st['txb'][sl] = tv * st['dxb'][sl]
            st['tyb'][sl] = tv * st['dyb'][sl]
            st['tzb'][sl] = tv * st['dzb'][sl]

    def issue_scat(s):
        st = S[s]
        pltpu.async_copy(st['mb'], msum.at[st['rowi']], st['semw'], add=True)
        pltpu.async_copy(st['txb'], nx_s.at[st['rowi']], st['semw'], add=True)
        pltpu.async_copy(st['tyb'], ny_s.at[st['rowi']], st['semw'], add=True)
        pltpu.async_copy(st['tzb'], nz_s.at[st['rowi']], st['semw'], add=True)
        pltpu.async_copy(onesb, cnt_s.at[st['rowi']], st['semw'], add=True)

    def wait_scat(s):
        st = S[s]
        pltpu.make_async_copy(st['mb'], msum.at[st['rowi']],
                              st['semw']).wait()
        pltpu.make_async_copy(st['txb'], nx_s.at[st['rowi']],
                              st['semw']).wait()
        pltpu.make_async_copy(st['tyb'], ny_s.at[st['rowi']],
                              st['semw']).wait()
        pltpu.make_async_copy(st['tzb'], nz_s.at[st['rowi']],
                              st['semw']).wait()
        pltpu.make_async_copy(onesb, cnt_s.at[st['rowi']],
                              st['semw']).wait()

    issue_reads(0, 0)

    def trip_body(j, carry):
        for b in range(3):
            k = 3 * j + b
            s = b

            @pl.when(k < nch)
            def _():
                @pl.when(k >= 2)
                def _():
                    wait_scat((b + 1) % 3)

                @pl.when(k + 1 < nch)
                def _():
                    issue_reads(k + 1, (b + 1) % 3)
                wait_reads(k, s)
                compute(s)
                issue_scat(s)
        return carry

    lax.fori_loop(0, (nch + 2) // 3, trip_body, 0)
    wait_scat((nch - 2) % 3)
    wait_scat((nch - 1) % 3)

    plsc.subcore_barrier()

    pltpu.sync_copy(msum.at[pl.ds(off1, c1), :],
                    msp_h.at[cid, pl.ds(off1, c1), :])
    for k, acc in enumerate((nx_s, ny_s, nz_s, cnt_s)):
        rb = pl.multiple_of((cid * 4 + k) * NPAD + off1, 8)
        pltpu.sync_copy(acc.at[pl.ds(off1, c1)], stg.at[pl.ds(0, c1)])
        pltpu.sync_copy(stg.at[pl.ds(0, c1)], np4_h.at[pl.ds(rb, c1)])

    @pl.when(sid == 0)
    def _():
        t0 = pl.multiple_of(c1 * NS, 8)
        if tail > 0:
            pltpu.sync_copy(msum.at[pl.ds(t0, tail), :],
                            msp_h.at[cid, pl.ds(t0, tail), :])
        for k, acc in enumerate((nx_s, ny_s, nz_s, cnt_s)):
            rb0 = pl.multiple_of((cid * 4 + k) * NPAD, 8)
            if tail > 0:
                pltpu.sync_copy(acc.at[pl.ds(t0, tail)], stg.at[pl.ds(0, tail)])
                pltpu.sync_copy(stg.at[pl.ds(0, tail)],
                                np4_h.at[pl.ds(rb0 + t0, tail)])
            if NPAD > N:
                pltpu.sync_copy(znum.at[pl.ds(0, NPAD - N)],
                                np4_h.at[pl.ds(rb0 + N, NPAD - N)])


def _tcd_body(s_ref, msp_ref, ct_ref, np4_ref,
              wp1a_ref, wp1b_ref, bp1_ref, wp2_ref, bp2_ref,
              so_ref, co_ref):
    s = s_ref[...]
    ms = msp_ref[0] + msp_ref[1]
    p = _silu(jnp.dot(s, wp1a_ref[...], preferred_element_type=jnp.float32)
              + jnp.dot(ms, wp1b_ref[...], preferred_element_type=jnp.float32)
              + bp1_ref[...])
    so_ref[...] = s + jnp.dot(p, wp2_ref[...],
                              preferred_element_type=jnp.float32) + bp2_ref[...]
    nps = np4_ref[0] + np4_ref[1]
    cnt = nps[3:4, :]
    co_ref[...] = ct_ref[...] + nps / jnp.clip(cnt, 1.0, None)


def kernel(edge_index, str_feature, coord_feature, W_m1, b_m1, W_m2, b_m2,
           W_t1, b_t1, W_t2, b_t2, W_t3, W_p1, b_p1, W_p2, b_p2):
    N, raw = str_feature.shape
    E = edge_index.shape[1]
    hid = W_m2.shape[0]
    f32 = jnp.float32
    NPAD = 10240
    assert E % (NW * CHUNK) == 0 and N % NS == 0

    row = edge_index[0]
    col = edge_index[1]
    cx = coord_feature[:, 0]
    cy = coord_feature[:, 1]
    cz = coord_feature[:, 2]

    W_aT = W_m1[:, :raw].T
    W_bT = W_m1[:, raw:2 * raw].T
    wc_row = W_m1[:, 2 * raw].reshape(1, hid)
    b1_row = b_m1.reshape(1, hid)
    Wm2T = W_m2.T.astype(jnp.bfloat16)
    b2_row = b_m2.reshape(1, hid)
    Wt1T = W_t1.T.astype(jnp.bfloat16)
    bt1_row = b_t1.reshape(1, hid)
    Wt2T = W_t2.T.astype(jnp.bfloat16)
    bt2_row = b_t2.reshape(1, hid)
    wt3_row = W_t3.reshape(1, hid)
    Wp1aT = W_p1[:, :raw].T
    Wp1bT = W_p1[:, raw:].T
    bp1_row = b_p1.reshape(1, hid)
    Wp2T = W_p2.T
    bp2_row = b_p2.reshape(1, raw)

    P, Q = pl.pallas_call(
        _tc0_body,
        out_shape=(jax.ShapeDtypeStruct((N, hid), f32),
                   jax.ShapeDtypeStruct((N, hid), f32)),
    )(str_feature, W_aT, W_bT, b1_row)

    mesh = plsc.VectorSubcoreMesh(core_axis_name="c", subcore_axis_name="s")
    sca = pl.kernel(
        functools.partial(_sca_body, E=E),
        compiler_params=pltpu.CompilerParams(needs_layout_passes=False),
        out_type=(jax.ShapeDtypeStruct((E, hid), f32),
                  jax.ShapeDtypeStruct((E,), f32),
                  jax.ShapeDtypeStruct((E,), f32),
                  jax.ShapeDtypeStruct((E,), f32),
                  jax.ShapeDtypeStruct((E,), f32)),
        mesh=mesh,
        scratch_types=(
            [pltpu.VMEM((CHUNK,), jnp.int32),
             pltpu.VMEM((CHUNK,), jnp.int32),
             pltpu.VMEM((CHUNK, hid), f32),
             pltpu.VMEM((CHUNK, hid), f32),
             pltpu.VMEM((CHUNK,), f32),
             pltpu.VMEM((CHUNK,), f32),
             pltpu.VMEM((CHUNK,), f32),
             pltpu.VMEM((CHUNK,), f32)] * 2
            + [pltpu.VMEM((N,), f32),
               pltpu.VMEM((N,), f32),
               pltpu.VMEM((N,), f32)]
            + [pltpu.SemaphoreType.DMA] * 6
        ),
    )
    z, dxe, dye, dze, d2e = sca(row, col, P, Q, cx, cy, cz)

    BE = 2560
    nb = E // BE
    tr = BE // 128
    msg, t2d = pl.pallas_call(
        _tcb_body,
        grid=(nb,),
        in_specs=[
            pl.BlockSpec((BE, hid), lambda i: (i, 0)),
            pl.BlockSpec((E // 128, 128), lambda i: (0, 0)),
            pl.BlockSpec((1, hid), lambda i: (0, 0)),
            pl.BlockSpec((hid, hid), lambda i: (0, 0)),
            pl.BlockSpec((1, hid), lambda i: (0, 0)),
            pl.BlockSpec((hid, hid), lambda i: (0, 0)),
            pl.BlockSpec((1, hid), lambda i: (0, 0)),
            pl.BlockSpec((hid, hid), lambda i: (0, 0)),
            pl.BlockSpec((1, hid), lambda i: (0, 0)),
            pl.BlockSpec((1, hid), lambda i: (0, 0)),
        ],
        out_specs=[
            pl.BlockSpec((BE, hid), lambda i: (i, 0)),
            pl.BlockSpec((E // 128, 128), lambda i: (0, 0)),
        ],
        out_shape=(jax.ShapeDtypeStruct((E, hid), f32),
                   jax.ShapeDtypeStruct((E // 128, 128), f32)),
    )(z, d2e.reshape(E // 128, 128), wc_row, Wm2T, b2_row,
      Wt1T, bt1_row, Wt2T, bt2_row, wt3_row)

    scc = pl.kernel(
        functools.partial(_scc_body, E=E, N=N, NPAD=NPAD),
        out_type=(jax.ShapeDtypeStruct((NC, N, hid), f32),
                  jax.ShapeDtypeStruct((NC * 4 * NPAD,), f32)),
        mesh=mesh,
        scratch_types=(
            [pltpu.VMEM((CHUNK,), jnp.int32),
             pltpu.VMEM((CHUNK, hid), f32),
             pltpu.VMEM((CHUNK,), f32),
             pltpu.VMEM((CHUNK,), f32),
             pltpu.VMEM((CHUNK,), f32),
             pltpu.VMEM((CHUNK,), f32),
             pltpu.VMEM((CHUNK,), f32),
             pltpu.VMEM((CHUNK,), f32),
             pltpu.VMEM((CHUNK,), f32),
             pltpu.SemaphoreType.DMA,
             pltpu.SemaphoreType.DMA] * 3
            + [pltpu.VMEM((CHUNK,), f32),
               pltpu.VMEM((104, hid), f32),
               pltpu.VMEM((640,), f32),
               pltpu.VMEM((640,), f32),
               pltpu.VMEM_SHARED((N, hid), f32),
               pltpu.VMEM_SHARED((N,), f32),
               pltpu.VMEM_SHARED((N,), f32),
               pltpu.VMEM_SHARED((N,), f32),
               pltpu.VMEM_SHARED((N,), f32)]
        ),
    )
    msp, np4 = scc(row, msg, t2d.reshape(E), dxe, dye, dze)
    np4 = np4.reshape(NC, 4, NPAD)

    coordT4 = jnp.zeros((4, NPAD), f32).at[:3, :N].set(coord_feature.T)
    BN = 2000
    BNP = NPAD // (N // BN)
    str_out, coT = pl.pallas_call(
        _tcd_body,
        grid=(N // BN,),
        in_specs=[
            pl.BlockSpec((BN, hid), lambda i: (i, 0)),
            pl.BlockSpec((NC, BN, hid), lambda i: (0, i, 0)),
            pl.BlockSpec((4, BNP), lambda i: (0, i)),
            pl.BlockSpec((NC, 4, BNP), lambda i: (0, 0, i)),
            pl.BlockSpec((raw, hid), lambda i: (0, 0)),
            pl.BlockSpec((hid, hid), lambda i: (0, 0)),
            pl.BlockSpec((1, hid), lambda i: (0, 0)),
            pl.BlockSpec((hid, raw), lambda i: (0, 0)),
            pl.BlockSpec((1, raw), lambda i: (0, 0)),
        ],
        out_specs=[
            pl.BlockSpec((BN, raw), lambda i: (i, 0)),
            pl.BlockSpec((4, BNP), lambda i: (0, i)),
        ],
        out_shape=(jax.ShapeDtypeStruct((N, raw), f32),
                   jax.ShapeDtypeStruct((4, NPAD), f32)),
    )(str_feature, msp, coordT4, np4, Wp1aT, Wp1bT, bp1_row, Wp2T, bp2_row)

    coord_out = coT[:3, :N].T
    return str_out, coord_out


# speedup vs baseline: 1.7337x; 1.0969x over previous
import functools

import jax
import jax.numpy as jnp
from jax import lax
from jax.experimental import pallas as pl
from jax.experimental.pallas import tpu as pltpu
from jax.experimental.pallas import tpu_sc as plsc

NC = 2
NS = 16
NW = NC * NS
CHUNK = 40


def _silu(x):
    return x * jax.nn.sigmoid(x)


def _g16(n):
    s = list(range(0, n - 15, 16))
    if s[-1] + 16 < n:
        s.append(n - 16)
    return s


def _tc0_body(s_ref, wa_ref, wb_ref, b1_ref, p_ref, q_ref):
    s = s_ref[...]
    p_ref[...] = jnp.dot(s, wa_ref[...],
                         preferred_element_type=jnp.float32) + b1_ref[...]
    q_ref[...] = jnp.dot(s, wb_ref[...], preferred_element_type=jnp.float32)


def _sca_body(row_h, col_h, p_h, q_h, cx_h, cy_h, cz_h,
              z_h, dx_h, dy_h, dz_h, d2_h,
              rowi0, coli0, zb0, qb0, dxb0, dyb0, dzb0, d2b0,
              rowi1, coli1, zb1, qb1, dxb1, dyb1, dzb1, d2b1,
              cxv, cyv, czv,
              semi0, semg0, semw0, semi1, semg1, semw1, E=None):
    cid = lax.axis_index("c")
    sid = lax.axis_index("s")
    wid = sid * NC + cid
    epw = E // NW
    nch = epw // CHUNK
    base0 = pl.multiple_of(wid * epw, 8)
    S = [
        dict(rowi=rowi0, coli=coli0, zb=zb0, qb=qb0, dxb=dxb0, dyb=dyb0,
             dzb=dzb0, d2b=d2b0, semi=semi0, semg=semg0, semw=semw0),
        dict(rowi=rowi1, coli=coli1, zb=zb1, qb=qb1, dxb=dxb1, dyb=dyb1,
             dzb=dzb1, d2b=d2b1, semi=semi1, semg=semg1, semw=semw1),
    ]

    def bse(k):
        return pl.multiple_of(base0 + k * CHUNK, 8)

    def issue_idx(k, s):
        pltpu.async_copy(row_h.at[pl.ds(bse(k), CHUNK)], S[s]['rowi'],
                         S[s]['semi'])
        pltpu.async_copy(col_h.at[pl.ds(bse(k), CHUNK)], S[s]['coli'],
                         S[s]['semi'])

    def wait_idx(k, s):
        pltpu.make_async_copy(row_h.at[pl.ds(bse(k), CHUNK)], S[s]['rowi'],
                              S[s]['semi']).wait()
        pltpu.make_async_copy(col_h.at[pl.ds(bse(k), CHUNK)], S[s]['coli'],
                              S[s]['semi']).wait()

    def issue_gat(s):
        pltpu.async_copy(p_h.at[S[s]['rowi']], S[s]['zb'], S[s]['semg'])
        pltpu.async_copy(q_h.at[S[s]['coli']], S[s]['qb'], S[s]['semg'])

    def wait_gat(s):
        pltpu.make_async_copy(p_h.at[S[s]['rowi']], S[s]['zb'],
                              S[s]['semg']).wait()
        pltpu.make_async_copy(q_h.at[S[s]['coli']], S[s]['qb'],
                              S[s]['semg']).wait()

    def geometry(s):
        st = S[s]
        for g0 in _g16(CHUNK):
            sl = pl.ds(g0, 16)
            rv = st['rowi'][sl]
            cv = st['coli'][sl]
            dx = plsc.load_gather(cxv, [rv]) - plsc.load_gather(cxv, [cv])
            dy = plsc.load_gather(cyv, [rv]) - plsc.load_gather(cyv, [cv])
            dz = plsc.load_gather(czv, [rv]) - plsc.load_gather(czv, [cv])
            st['dxb'][sl] = dx
            st['dyb'][sl] = dy
            st['dzb'][sl] = dz
            st['d2b'][sl] = dx * dx + dy * dy + dz * dz

    def add_q(s):
        st = S[s]

        def add_row(e, c):
            for j in range(8):
                plsc.addupdate(st['zb'].at[e, pl.ds(j * 16, 16)],
                               st['qb'][e, pl.ds(j * 16, 16)])
            return c
        lax.fori_loop(0, CHUNK, add_row, 0, unroll=2)

    def issue_outs(k, s):
        st = S[s]
        pltpu.async_copy(st['zb'], z_h.at[pl.ds(bse(k), CHUNK), :], st['semw'])
        pltpu.async_copy(st['dxb'], dx_h.at[pl.ds(bse(k), CHUNK)], st['semw'])
        pltpu.async_copy(st['dyb'], dy_h.at[pl.ds(bse(k), CHUNK)], st['semw'])
        pltpu.async_copy(st['dzb'], dz_h.at[pl.ds(bse(k), CHUNK)], st['semw'])
        pltpu.async_copy(st['d2b'], d2_h.at[pl.ds(bse(k), CHUNK)], st['semw'])

    def wait_outs(k, s):
        st = S[s]
        pltpu.make_async_copy(st['zb'], z_h.at[pl.ds(bse(k), CHUNK), :],
                              st['semw']).wait()
        pltpu.make_async_copy(st['dxb'], dx_h.at[pl.ds(bse(k), CHUNK)],
                              st['semw']).wait()
        pltpu.make_async_copy(st['dyb'], dy_h.at[pl.ds(bse(k), CHUNK)],
                              st['semw']).wait()
        pltpu.make_async_copy(st['dzb'], dz_h.at[pl.ds(bse(k), CHUNK)],
                              st['semw']).wait()
        pltpu.make_async_copy(st['d2b'], d2_h.at[pl.ds(bse(k), CHUNK)],
                              st['semw']).wait()

    pltpu.sync_copy(cx_h, cxv)
    pltpu.sync_copy(cy_h, cyv)
    pltpu.sync_copy(cz_h, czv)

    issue_idx(0, 0)
    wait_idx(0, 0)
    issue_gat(0)
    issue_idx(1, 1)

    def pair_body(j, carry):
        for b in range(2):
            k = 2 * j + b
            s = b
            o = 1 - b
            wait_idx(k + 1, o)

            @pl.when(k >= 1)
            def _():
                wait_outs(k - 1, o)
            issue_gat(o)
            geometry(s)
            wait_gat(s)

            @pl.when(k <= nch - 3)
            def _():
                issue_idx(k + 2, s)
            add_q(s)
            issue_outs(k, s)
        return carry

    lax.fori_loop(0, (nch - 1) // 2, pair_body, 0)

    k = nch - 1
    wait_outs(k - 1, 1)
    geometry(0)
    wait_gat(0)
    add_q(0)
    issue_outs(k, 0)
    wait_outs(k, 0)


def _tcb_body(z_ref, d2_ref, wc_ref, wm2_ref, b2_ref,
              wt1_ref, bt1_ref, wt2_ref, bt2_ref, wt3_ref,
              msg_ref, t_ref):
    bf16 = jnp.bfloat16
    ntr = z_ref.shape[0] // 128
    ib = pl.program_id(0)
    wc = wc_ref[...]
    rows = []
    for r in range(ntr):
        d2r = d2_ref[pl.ds(ib * ntr + r, 1), :]
        rows.append(lax.dot_general(d2r, wc, (((0,), (0,)), ((), ())),
                                    preferred_element_type=jnp.float32))
    u = _silu(z_ref[...] + jnp.concatenate(rows, axis=0))
    m = _silu(jnp.dot(u.astype(bf16), wm2_ref[...],
                      preferred_element_type=jnp.float32) + b2_ref[...])
    t1 = _silu(jnp.dot(m.astype(bf16), wt1_ref[...],
                       preferred_element_type=jnp.float32) + bt1_ref[...])
    t2 = _silu(jnp.dot(t1.astype(bf16), wt2_ref[...],
                       preferred_element_type=jnp.float32) + bt2_ref[...])
    msg_ref[...] = m
    tv = jnp.sum(t2 * wt3_ref[...], axis=1)
    tr = t2.shape[0] // 128
    i = pl.program_id(0)
    t_ref[pl.ds(i * tr, tr), :] = jnp.reshape(tv, (tr, 128))


def _scc_body(row_h, msg_h, t_h, dx_h, dy_h, dz_h,
              msp_h, np4_h,
              rowi0, mb0, tb0, dxb0, dyb0, dzb0, txb0, tyb0, tzb0,
              semr0, semw0,
              rowi1, mb1, tb1, dxb1, dyb1, dzb1, txb1, tyb1, tzb1,
              semr1, semw1,
              rowi2, mb2, tb2, dxb2, dyb2, dzb2, txb2, tyb2, tzb2,
              semr2, semw2,
              onesb, zrow, znum, stg, msum, nx_s, ny_s, nz_s, cnt_s,
              E=None, N=None, NPAD=None):
    S = [
        dict(rowi=rowi0, mb=mb0, tb=tb0, dxb=dxb0, dyb=dyb0, dzb=dzb0,
             txb=txb0, tyb=tyb0, tzb=tzb0, semr=semr0, semw=semw0),
        dict(rowi=rowi1, mb=mb1, tb=tb1, dxb=dxb1, dyb=dyb1, dzb=dzb1,
             txb=txb1, tyb=tyb1, tzb=tzb1, semr=semr1, semw=semw1),
        dict(rowi=rowi2, mb=mb2, tb=tb2, dxb=dxb2, dyb=dyb2, dzb=dzb2,
             txb=txb2, tyb=tyb2, tzb=tzb2, semr=semr2, semw=semw2),
    ]
    cid = lax.axis_index("c")
    sid = lax.axis_index("s")
    wid = sid * NC + cid
    epw = E // NW
    nch = epw // CHUNK
    base0 = pl.multiple_of(wid * epw, 8)
    c1 = (N // NS // 8) * 8
    tail = N - c1 * NS
    off1 = pl.multiple_of(sid * c1, 8)

    zv = jnp.zeros((16,), jnp.float32)

    def zr_body(r, c):
        for j in range(8):
            zrow[r, pl.ds(j * 16, 16)] = zv
        return c
    lax.fori_loop(0, zrow.shape[0], zr_body, 0, unroll=2)

    def zn_body(r, c):
        znum[pl.ds(r * 16, 16)] = zv
        return c
    lax.fori_loop(0, znum.shape[0] // 16, zn_body, 0, unroll=2)

    zr = zrow.shape[0]
    for k in range(c1 // zr):
        pltpu.sync_copy(zrow, msum.at[pl.ds(off1 + k * zr, zr), :])
    for acc in (nx_s, ny_s, nz_s, cnt_s):
        pltpu.sync_copy(znum.at[pl.ds(0, c1)], acc.at[pl.ds(off1, c1)])

    if tail > 0:
        @pl.when(sid == 0)
        def _():
            t0 = pl.multiple_of(c1 * NS, 8)
            pltpu.sync_copy(zrow.at[pl.ds(0, tail), :],
                            msum.at[pl.ds(t0, tail), :])
            for acc in (nx_s, ny_s, nz_s, cnt_s):
                pltpu.sync_copy(znum.at[pl.ds(0, tail)],
                                acc.at[pl.ds(t0, tail)])

    for g0 in _g16(CHUNK):
        onesb[pl.ds(g0, 16)] = jnp.ones((16,), jnp.float32)

    plsc.subcore_barrier()

    def bse(k):
        return pl.multiple_of(base0 + k * CHUNK, 8)

    def issue_reads(k, s):
        st = S[s]
        pltpu.async_copy(row_h.at[pl.ds(bse(k), CHUNK)], st['rowi'],
                         st['semr'])
        pltpu.async_copy(msg_h.at[pl.ds(bse(k), CHUNK), :], st['mb'],
                         st['semr'])
        pltpu.async_copy(t_h.at[pl.ds(bse(k), CHUNK)], st['tb'], st['semr'])
        pltpu.async_copy(dx_h.at[pl.ds(bse(k), CHUNK)], st['dxb'], st['semr'])
        pltpu.async_copy(dy_h.at[pl.ds(bse(k), CHUNK)], st['dyb'], st['semr'])
        pltpu.async_copy(dz_h.at[pl.ds(bse(k), CHUNK)], st['dzb'], st['semr'])

    def wait_reads(k, s):
        st = S[s]
        pltpu.make_async_copy(row_h.at[pl.ds(bse(k), CHUNK)], st['rowi'],
                              st['semr']).wait()
        pltpu.make_async_copy(msg_h.at[pl.ds(bse(k), CHUNK), :], st['mb'],
                              st['semr']).wait()
        pltpu.make_async_copy(t_h.at[pl.ds(bse(k), CHUNK)], st['tb'],
                              st['semr']).wait()
        pltpu.make_async_copy(dx_h.at[pl.ds(bse(k), CHUNK)], st['dxb'],
                              st['semr']).wait()
        pltpu.make_async_copy(dy_h.at[pl.ds(bse(k), CHUNK)], st['dyb'],
                              st['semr']).wait()
        pltpu.make_async_copy(dz_h.at[pl.ds(bse(k), CHUNK)], st['dzb'],
                              st['semr']).wait()

    def compute(s):
        st = S[s]
        for g0 in _g16(CHUNK):
            sl = pl.ds(g0, 16)
            tv = st['tb'][sl]
            st['txb'][sl] = tv * st['dxb'][sl]
            st['tyb'][sl] = tv * st['dyb'][sl]
            st['tzb'][sl] = tv * st['dzb'][sl]

    def issue_scat(s):
        st = S[s]
        pltpu.async_copy(st['mb'], msum.at[st['rowi']], st['semw'], add=True)
        pltpu.async_copy(st['txb'], nx_s.at[st['rowi']], st['semw'], add=True)
        pltpu.async_copy(st['tyb'], ny_s.at[st['rowi']], st['semw'], add=True)
        pltpu.async_copy(st['tzb'], nz_s.at[st['rowi']], st['semw'], add=True)
        pltpu.async_copy(onesb, cnt_s.at[st['rowi']], st['semw'], add=True)

    def wait_scat(s):
        st = S[s]
        pltpu.make_async_copy(st['mb'], msum.at[st['rowi']],
                              st['semw']).wait()
        pltpu.make_async_copy(st['txb'], nx_s.at[st['rowi']],
                              st['semw']).wait()
        pltpu.make_async_copy(st['tyb'], ny_s.at[st['rowi']],
                              st['semw']).wait()
        pltpu.make_async_copy(st['tzb'], nz_s.at[st['rowi']],
                              st['semw']).wait()
        pltpu.make_async_copy(onesb, cnt_s.at[st['rowi']],
                              st['semw']).wait()

    issue_reads(0, 0)

    def trip_body(j, carry):
        for b in range(3):
            k = 3 * j + b
            s = b

            @pl.when(k < nch)
            def _():
                @pl.when(k >= 2)
                def _():
                    wait_scat((b + 1) % 3)

                @pl.when(k + 1 < nch)
                def _():
                    issue_reads(k + 1, (b + 1) % 3)
                wait_reads(k, s)
                compute(s)
                issue_scat(s)
        return carry

    lax.fori_loop(0, (nch + 2) // 3, trip_body, 0)
    wait_scat((nch - 2) % 3)
    wait_scat((nch - 1) % 3)

    plsc.subcore_barrier()

    pltpu.sync_copy(msum.at[pl.ds(off1, c1), :],
                    msp_h.at[cid, pl.ds(off1, c1), :])
    for k, acc in enumerate((nx_s, ny_s, nz_s, cnt_s)):
        rb = pl.multiple_of((cid * 4 + k) * NPAD + off1, 8)
        pltpu.sync_copy(acc.at[pl.ds(off1, c1)], stg.at[pl.ds(0, c1)])
        pltpu.sync_copy(stg.at[pl.ds(0, c1)], np4_h.at[pl.ds(rb, c1)])

    @pl.when(sid == 0)
    def _():
        t0 = pl.multiple_of(c1 * NS, 8)
        if tail > 0:
            pltpu.sync_copy(msum.at[pl.ds(t0, tail), :],
                            msp_h.at[cid, pl.ds(t0, tail), :])
        for k, acc in enumerate((nx_s, ny_s, nz_s, cnt_s)):
            rb0 = pl.multiple_of((cid * 4 + k) * NPAD, 8)
            if tail > 0:
                pltpu.sync_copy(acc.at[pl.ds(t0, tail)], stg.at[pl.ds(0, tail)])
                pltpu.sync_copy(stg.at[pl.ds(0, tail)],
                                np4_h.at[pl.ds(rb0 + t0, tail)])
            if NPAD > N:
                pltpu.sync_copy(znum.at[pl.ds(0, NPAD - N)],
                                np4_h.at[pl.ds(rb0 + N, NPAD - N)])


def _tcd_body(s_ref, mspa_ref, mspb_ref, ct_ref, np4a_ref, np4b_ref,
              wp1a_ref, wp1b_ref, bp1_ref, wp2_ref, bp2_ref,
              so_ref, co_ref):
    s = s_ref[...]
    ms = (mspa_ref[0] + mspa_ref[1]) + (mspb_ref[0] + mspb_ref[1])
    p = _silu(jnp.dot(s, wp1a_ref[...], preferred_element_type=jnp.float32)
              + jnp.dot(ms, wp1b_ref[...], preferred_element_type=jnp.float32)
              + bp1_ref[...])
    so_ref[...] = s + jnp.dot(p, wp2_ref[...],
                              preferred_element_type=jnp.float32) + bp2_ref[...]
    nps = (np4a_ref[0] + np4a_ref[1]) + (np4b_ref[0] + np4b_ref[1])
    cnt = nps[3:4, :]
    co_ref[...] = ct_ref[...] + nps / jnp.clip(cnt, 1.0, None)


def kernel(edge_index, str_feature, coord_feature, W_m1, b_m1, W_m2, b_m2,
           W_t1, b_t1, W_t2, b_t2, W_t3, W_p1, b_p1, W_p2, b_p2):
    N, raw = str_feature.shape
    E = edge_index.shape[1]
    hid = W_m2.shape[0]
    f32 = jnp.float32
    NPAD = 10240
    assert E % (NW * CHUNK) == 0 and N % NS == 0

    row = edge_index[0]
    col = edge_index[1]
    cx = coord_feature[:, 0]
    cy = coord_feature[:, 1]
    cz = coord_feature[:, 2]

    W_aT = W_m1[:, :raw].T
    W_bT = W_m1[:, raw:2 * raw].T
    wc_row = W_m1[:, 2 * raw].reshape(1, hid)
    b1_row = b_m1.reshape(1, hid)
    Wm2T = W_m2.T.astype(jnp.bfloat16)
    b2_row = b_m2.reshape(1, hid)
    Wt1T = W_t1.T.astype(jnp.bfloat16)
    bt1_row = b_t1.reshape(1, hid)
    Wt2T = W_t2.T.astype(jnp.bfloat16)
    bt2_row = b_t2.reshape(1, hid)
    wt3_row = W_t3.reshape(1, hid)
    Wp1aT = W_p1[:, :raw].T
    Wp1bT = W_p1[:, raw:].T
    bp1_row = b_p1.reshape(1, hid)
    Wp2T = W_p2.T
    bp2_row = b_p2.reshape(1, raw)

    P, Q = pl.pallas_call(
        _tc0_body,
        out_shape=(jax.ShapeDtypeStruct((N, hid), f32),
                   jax.ShapeDtypeStruct((N, hid), f32)),
    )(str_feature, W_aT, W_bT, b1_row)

    EH = E // 2
    mesh = plsc.VectorSubcoreMesh(core_axis_name="c", subcore_axis_name="s")
    sca = pl.kernel(
        functools.partial(_sca_body, E=EH),
        compiler_params=pltpu.CompilerParams(needs_layout_passes=False),
        out_type=(jax.ShapeDtypeStruct((EH, hid), f32),
                  jax.ShapeDtypeStruct((EH,), f32),
                  jax.ShapeDtypeStruct((EH,), f32),
                  jax.ShapeDtypeStruct((EH,), f32),
                  jax.ShapeDtypeStruct((EH,), f32)),
        mesh=mesh,
        scratch_types=(
            [pltpu.VMEM((CHUNK,), jnp.int32),
             pltpu.VMEM((CHUNK,), jnp.int32),
             pltpu.VMEM((CHUNK, hid), f32),
             pltpu.VMEM((CHUNK, hid), f32),
             pltpu.VMEM((CHUNK,), f32),
             pltpu.VMEM((CHUNK,), f32),
             pltpu.VMEM((CHUNK,), f32),
             pltpu.VMEM((CHUNK,), f32)] * 2
            + [pltpu.VMEM((N,), f32),
               pltpu.VMEM((N,), f32),
               pltpu.VMEM((N,), f32)]
            + [pltpu.SemaphoreType.DMA] * 6
        ),
    )

    BE = 3200
    nb = EH // BE

    def tcb(z, d2e):
        return pl.pallas_call(
            _tcb_body,
            grid=(nb,),
            in_specs=[
                pl.BlockSpec((BE, hid), lambda i: (i, 0)),
                pl.BlockSpec((EH // 128, 128), lambda i: (0, 0)),
                pl.BlockSpec((1, hid), lambda i: (0, 0)),
                pl.BlockSpec((hid, hid), lambda i: (0, 0)),
                pl.BlockSpec((1, hid), lambda i: (0, 0)),
                pl.BlockSpec((hid, hid), lambda i: (0, 0)),
                pl.BlockSpec((1, hid), lambda i: (0, 0)),
                pl.BlockSpec((hid, hid), lambda i: (0, 0)),
                pl.BlockSpec((1, hid), lambda i: (0, 0)),
                pl.BlockSpec((1, hid), lambda i: (0, 0)),
            ],
            out_specs=[
                pl.BlockSpec((BE, hid), lambda i: (i, 0)),
                pl.BlockSpec((EH // 128, 128), lambda i: (0, 0)),
            ],
            out_shape=(jax.ShapeDtypeStruct((EH, hid), f32),
                       jax.ShapeDtypeStruct((EH // 128, 128), f32)),
        )(z, d2e.reshape(EH // 128, 128), wc_row, Wm2T, b2_row,
          Wt1T, bt1_row, Wt2T, bt2_row, wt3_row)

    scc = pl.kernel(
        functools.partial(_scc_body, E=EH, N=N, NPAD=NPAD),
        out_type=(jax.ShapeDtypeStruct((NC, N, hid), f32),
                  jax.ShapeDtypeStruct((NC * 4 * NPAD,), f32)),
        mesh=mesh,
        scratch_types=(
            [pltpu.VMEM((CHUNK,), jnp.int32),
             pltpu.VMEM((CHUNK, hid), f32),
             pltpu.VMEM((CHUNK,), f32),
             pltpu.VMEM((CHUNK,), f32),
             pltpu.VMEM((CHUNK,), f32),
             pltpu.VMEM((CHUNK,), f32),
             pltpu.VMEM((CHUNK,), f32),
             pltpu.VMEM((CHUNK,), f32),
             pltpu.VMEM((CHUNK,), f32),
             pltpu.SemaphoreType.DMA,
             pltpu.SemaphoreType.DMA] * 3
            + [pltpu.VMEM((CHUNK,), f32),
               pltpu.VMEM((104, hid), f32),
               pltpu.VMEM((640,), f32),
               pltpu.VMEM((640,), f32),
               pltpu.VMEM_SHARED((N, hid), f32),
               pltpu.VMEM_SHARED((N,), f32),
               pltpu.VMEM_SHARED((N,), f32),
               pltpu.VMEM_SHARED((N,), f32),
               pltpu.VMEM_SHARED((N,), f32)]
        ),
    )
    msps, np4s = [], []
    for h in range(2):
        rh = lax.slice_in_dim(row, h * EH, (h + 1) * EH)
        ch = lax.slice_in_dim(col, h * EH, (h + 1) * EH)
        z, dxe, dye, dze, d2e = sca(rh, ch, P, Q, cx, cy, cz)
        msg, t2d = tcb(z, d2e)
        msp, np4 = scc(rh, msg, t2d.reshape(EH), dxe, dye, dze)
        msps.append(msp)
        np4s.append(np4.reshape(NC, 4, NPAD))

    coordT4 = jnp.zeros((4, NPAD), f32).at[:3, :N].set(coord_feature.T)
    BN = 2000
    BNP = NPAD // (N // BN)
    str_out, coT = pl.pallas_call(
        _tcd_body,
        grid=(N // BN,),
        in_specs=[
            pl.BlockSpec((BN, hid), lambda i: (i, 0)),
            pl.BlockSpec((NC, BN, hid), lambda i: (0, i, 0)),
            pl.BlockSpec((NC, BN, hid), lambda i: (0, i, 0)),
            pl.BlockSpec((4, BNP), lambda i: (0, i)),
            pl.BlockSpec((NC, 4, BNP), lambda i: (0, 0, i)),
            pl.BlockSpec((NC, 4, BNP), lambda i: (0, 0, i)),
            pl.BlockSpec((raw, hid), lambda i: (0, 0)),
            pl.BlockSpec((hid, hid), lambda i: (0, 0)),
            pl.BlockSpec((1, hid), lambda i: (0, 0)),
            pl.BlockSpec((hid, raw), lambda i: (0, 0)),
            pl.BlockSpec((1, raw), lambda i: (0, 0)),
        ],
        out_specs=[
            pl.BlockSpec((BN, raw), lambda i: (i, 0)),
            pl.BlockSpec((4, BNP), lambda i: (0, i)),
        ],
        out_shape=(jax.ShapeDtypeStruct((N, raw), f32),
                   jax.ShapeDtypeStruct((4, NPAD), f32)),
    )(str_feature, msps[0], msps[1], coordT4, np4s[0], np4s[1],
      Wp1aT, Wp1bT, bp1_row, Wp2T, bp2_row)

    coord_out = coT[:3, :N].T
    return str_out, coord_out
